# baseline (speedup 1.0000x reference)
import functools

import jax
import jax.numpy as jnp
from jax import lax
from jax.experimental import pallas as pl
from jax.experimental.pallas import tpu as pltpu
from jax.experimental.pallas import tpu_sc as plsc

N = 10000
NPAD = 10240
DIN = 128
DH = 128
DOUT = 40
DOP = 128
E = 320000
NCORES = 2
NSUB = 16
NTILES = NCORES * NSUB
CH = 32
TCH = 10240
EPAD = TCH * CH
RPS = NPAD // NSUB
NCHQ = 40
NBUF = 8
NCH0 = 320
NCH1 = 640 - NCH0
CHD = 128
TCHD = EPAD // CHD
NCHD = TCHD // NTILES
NCHQD = 40
DDEG = 128
DINV = 16

_MESH = plsc.VectorSubcoreMesh(core_axis_name="c", subcore_axis_name="s")

import numpy as _np

_PAD_I = _np.arange(EPAD - E, dtype=_np.int32)
_PAD_SRC = _PAD_I % N
_PAD_DST = N + _PAD_I % (NPAD - N)


def _fill_rows(ref, rows, width, value):
    @pl.loop(0, rows)
    def _(r):
        @pl.loop(0, width, step=16)
        def _(cc):
            ref[r, pl.ds(cc, 16)] = jnp.full((16,), value, jnp.float32)


def _sc_degree(dst3):
    @functools.partial(
        pl.kernel,
        out_type=jax.ShapeDtypeStruct((NCORES, NPAD, DDEG), jnp.float32),
        mesh=_MESH,
        scratch_types=[
            pltpu.VMEM((NCHQD, CHD), jnp.int32),
            pltpu.VMEM((CHD, DDEG), jnp.float32),
            pltpu.VMEM_SHARED((NPAD, DDEG), jnp.float32),
            [pltpu.SemaphoreType.DMA for _ in range(NBUF)],
        ],
    )
    def k(dst_hbm, out_hbm, dst_v, val_v, acc, ssem):
        c = lax.axis_index("c")
        s = lax.axis_index("s")
        wid = c * NSUB + s
        _fill_rows(val_v, CHD, DDEG, 0.0)

        @pl.loop(0, RPS // CHD)
        def _(i):
            pltpu.sync_copy(val_v, acc.at[pl.ds(s * RPS + i * CHD, CHD)])

        _fill_rows(val_v, CHD, DDEG, 1.0)
        plsc.subcore_barrier()

        @pl.loop(0, NCHD // NCHQD)
        def _(q):
            off = pl.multiple_of(wid * NCHD + q * NCHQD, NCHQD)
            pltpu.sync_copy(dst_hbm.at[pl.ds(off, NCHQD)], dst_v)
            for b in range(NBUF):
                pltpu.async_copy(
                    val_v, acc.at[dst_v.at[b]], ssem[b], add=True
                )

            @pl.loop(0, NCHQD - NBUF, step=NBUF)
            def _(k0):
                for b in range(NBUF):
                    pltpu.make_async_copy(
                        val_v, acc.at[dst_v.at[k0 + b]], ssem[b]
                    ).wait()
                    pltpu.async_copy(
                        val_v, acc.at[dst_v.at[k0 + NBUF + b]], ssem[b],
                        add=True,
                    )

            for b in range(NBUF):
                pltpu.make_async_copy(
                    val_v, acc.at[dst_v.at[NCHQD - NBUF + b]], ssem[b]
                ).wait()

        plsc.subcore_barrier()
        pltpu.sync_copy(
            acc.at[pl.ds(s * RPS, RPS)], out_hbm.at[c, pl.ds(s * RPS, RPS)]
        )

    return k(dst3)


def _sc_aggregate(table, src3, dst3, d):
    @functools.partial(
        pl.kernel,
        out_type=jax.ShapeDtypeStruct((NCORES, NPAD, d), jnp.float32),
        mesh=_MESH,
        scratch_types=[
            pltpu.VMEM((NCHQ, CH), jnp.int32),
            pltpu.VMEM((NCHQ, CH), jnp.int32),
            [pltpu.VMEM((CH, d), jnp.float32) for _ in range(NBUF)],
            pltpu.VMEM_SHARED((NPAD, d), jnp.float32),
            [pltpu.SemaphoreType.DMA for _ in range(NBUF)],
            [pltpu.SemaphoreType.DMA for _ in range(NBUF)],
        ],
    )
    def k(tab_hbm, src_hbm, dst_hbm, out_hbm, src_v, dst_v, bufs, acc,
          gsem, ssem):
        c = lax.axis_index("c")
        s = lax.axis_index("s")
        _fill_rows(bufs[0], CH, d, 0.0)

        @pl.loop(0, RPS // CH)
        def _(i):
            pltpu.sync_copy(bufs[0], acc.at[pl.ds(s * RPS + i * CH, CH)])

        plsc.subcore_barrier()

        n_my = jnp.where(c == 0, NCH0, NCH1)
        base = c * (NSUB * NCH0) + s * n_my

        @pl.loop(0, n_my // NCHQ)
        def _(q):
            off = pl.multiple_of(base + q * NCHQ, NCHQ)
            pltpu.sync_copy(src_hbm.at[pl.ds(off, NCHQ)], src_v)
            pltpu.sync_copy(dst_hbm.at[pl.ds(off, NCHQ)], dst_v)
            for b in range(NBUF):
                pltpu.async_copy(tab_hbm.at[src_v.at[b]], bufs[b], gsem[b])

            @pl.loop(0, NCHQ - NBUF, step=NBUF)
            def _(k0):
                for b in range(NBUF):
                    pltpu.make_async_copy(
                        tab_hbm.at[src_v.at[k0 + b]], bufs[b], gsem[b]
                    ).wait()
                    pltpu.async_copy(
                        bufs[b], acc.at[dst_v.at[k0 + b]], ssem[b], add=True
                    )
                for b in range(NBUF):
                    pltpu.make_async_copy(
                        bufs[b], acc.at[dst_v.at[k0 + b]], ssem[b]
                    ).wait()
                    pltpu.async_copy(
                        tab_hbm.at[src_v.at[k0 + NBUF + b]], bufs[b], gsem[b]
                    )

            for b in range(NBUF):
                pltpu.make_async_copy(
                    tab_hbm.at[src_v.at[NCHQ - NBUF + b]], bufs[b], gsem[b]
                ).wait()
                pltpu.async_copy(
                    bufs[b], acc.at[dst_v.at[NCHQ - NBUF + b]], ssem[b],
                    add=True,
                )
            for b in range(NBUF):
                pltpu.make_async_copy(
                    bufs[b], acc.at[dst_v.at[NCHQ - NBUF + b]], ssem[b]
                ).wait()

        plsc.subcore_barrier()
        pltpu.sync_copy(
            acc.at[pl.ds(s * RPS, RPS)], out_hbm.at[c, pl.ds(s * RPS, RPS)]
        )

    return k(table, src3, dst3)


_R = 1024


def _tc_matmul(x_p, W1):
    def body(x_ref, w_ref, h_ref):
        h_ref[...] = jnp.dot(
            x_ref[...], w_ref[...], preferred_element_type=jnp.float32
        )

    return pl.pallas_call(
        body,
        grid=(NPAD // _R,),
        in_specs=[
            pl.BlockSpec((_R, DIN), lambda i: (i, 0)),
            pl.BlockSpec((DIN, DH), lambda i: (0, 0)),
        ],
        out_specs=pl.BlockSpec((_R, DH), lambda i: (i, 0)),
        out_shape=jax.ShapeDtypeStruct((NPAD, DH), jnp.float32),
    )(x_p, W1)


def _tc_pre(h, p16):
    def body(h_ref, p_ref, hs_ref, dinv_ref):
        deg16 = p_ref[0, :, :DINV] + p_ref[1, :, :DINV] + 1.0
        dinv16 = lax.rsqrt(deg16)
        hs_ref[...] = h_ref[...] * dinv16[:, 0:1]
        dinv_ref[...] = dinv16

    return pl.pallas_call(
        body,
        grid=(NPAD // _R,),
        in_specs=[
            pl.BlockSpec((_R, DH), lambda i: (i, 0)),
            pl.BlockSpec((NCORES, _R, DDEG), lambda i: (0, i, 0)),
        ],
        out_specs=[
            pl.BlockSpec((_R, DH), lambda i: (i, 0)),
            pl.BlockSpec((_R, DINV), lambda i: (i, 0)),
        ],
        out_shape=[
            jax.ShapeDtypeStruct((NPAD, DH), jnp.float32),
            jax.ShapeDtypeStruct((NPAD, DINV), jnp.float32),
        ],
    )(h, p16)


def _tc_mid(p1, hs1, dinv16, W2p, b1r):
    def body(p_ref, hs_ref, dinv_ref, w_ref, b_ref, out_ref):
        dinv = dinv_ref[:, 0:1]
        agg = p_ref[0] + p_ref[1] + hs_ref[...]
        out1 = agg * dinv + b_ref[...]
        r = jnp.maximum(out1, 0.0)
        h2 = jnp.dot(r, w_ref[...], preferred_element_type=jnp.float32)
        out_ref[...] = h2 * dinv

    return pl.pallas_call(
        body,
        grid=(NPAD // _R,),
        in_specs=[
            pl.BlockSpec((NCORES, _R, DH), lambda i: (0, i, 0)),
            pl.BlockSpec((_R, DH), lambda i: (i, 0)),
            pl.BlockSpec((_R, DINV), lambda i: (i, 0)),
            pl.BlockSpec((DH, DOP), lambda i: (0, 0)),
            pl.BlockSpec((1, DH), lambda i: (0, 0)),
        ],
        out_specs=pl.BlockSpec((_R, DOP), lambda i: (i, 0)),
        out_shape=jax.ShapeDtypeStruct((NPAD, DOP), jnp.float32),
    )(p1, hs1, dinv16, W2p, b1r)


def _tc_post(p2, hs2, dinv16, b2r):
    def body(p_ref, hs_ref, dinv_ref, b_ref, out_ref):
        dinv = dinv_ref[:, 0:1]
        logits = (p_ref[0] + p_ref[1] + hs_ref[...]) * dinv + b_ref[...]
        colmask = lax.broadcasted_iota(jnp.int32, (_R, DOP), 1) < DOUT
        lm = jnp.where(colmask, logits, jnp.float32(-1e30))
        m = jnp.max(lm, axis=1, keepdims=True)
        e = jnp.where(colmask, jnp.exp(logits - m), 0.0)
        ssum = jnp.sum(e, axis=1, keepdims=True)
        out_ref[...] = logits - m - jnp.log(ssum)

    return pl.pallas_call(
        body,
        grid=(NPAD // _R,),
        in_specs=[
            pl.BlockSpec((NCORES, _R, DOP), lambda i: (0, i, 0)),
            pl.BlockSpec((_R, DOP), lambda i: (i, 0)),
            pl.BlockSpec((_R, DINV), lambda i: (i, 0)),
            pl.BlockSpec((1, DOP), lambda i: (0, 0)),
        ],
        out_specs=pl.BlockSpec((_R, DOP), lambda i: (i, 0)),
        out_shape=jax.ShapeDtypeStruct((NPAD, DOP), jnp.float32),
    )(p2, hs2, dinv16, b2r)


def kernel(x, edge_index, W1, b1, W2, b2):
    src = edge_index[0]
    dst = edge_index[1]
    srcp = jnp.concatenate([src, jnp.asarray(_PAD_SRC)]).reshape(TCH, CH)
    dstp = jnp.concatenate([dst, jnp.asarray(_PAD_DST)]).reshape(TCH, CH)
    x_p = jnp.pad(x, ((0, NPAD - N), (0, 0)))
    W2p = jnp.pad(W2, ((0, 0), (0, DOP - DOUT)))
    b2r = jnp.pad(b2, ((0, DOP - DOUT),)).reshape(1, DOP)
    b1r = b1.reshape(1, DH)

    h1 = _tc_matmul(x_p, W1)
    p16 = _sc_degree(dstp.reshape(TCHD, CHD))
    hs1, dinv16 = _tc_pre(h1, p16)
    p1 = _sc_aggregate(hs1, srcp, dstp, DH)
    hs2 = _tc_mid(p1, hs1, dinv16, W2p, b1r)
    p2 = _sc_aggregate(hs2, srcp, dstp, DOP)
    out = _tc_post(p2, hs2, dinv16, b2r)
    return out[:N, :DOUT]


# speedup vs baseline: 1.0415x; 1.0415x over previous
import functools

import jax
import jax.numpy as jnp
from jax import lax
from jax.experimental import pallas as pl
from jax.experimental.pallas import tpu as pltpu
from jax.experimental.pallas import tpu_sc as plsc

N = 10000
NPAD = 10240
DIN = 128
DH = 128
DOUT = 40
DOP = 128
E = 320000
NCORES = 2
NSUB = 16
NTILES = NCORES * NSUB
CH = 64
TCH = 5120
EPAD = TCH * CH
RPS = NPAD // NSUB
NCHQ = 40
NBUF = 4
NCH0 = 160
NCH1 = 320 - NCH0
CHD = 128
TCHD = EPAD // CHD
NCHD = TCHD // NTILES
NCHQD = 40
DDEG = 128
DINV = 16

_MESH = plsc.VectorSubcoreMesh(core_axis_name="c", subcore_axis_name="s")

import numpy as _np

_PAD_I = _np.arange(EPAD - E, dtype=_np.int32)
_PAD_SRC = _PAD_I % N
_PAD_DST = N + _PAD_I % (NPAD - N)


def _fill_rows(ref, rows, width, value):
    @pl.loop(0, rows)
    def _(r):
        @pl.loop(0, width, step=16)
        def _(cc):
            ref[r, pl.ds(cc, 16)] = jnp.full((16,), value, jnp.float32)


def _sc_degree(dst3):
    @functools.partial(
        pl.kernel,
        out_type=jax.ShapeDtypeStruct((NCORES, NPAD, DDEG), jnp.float32),
        mesh=_MESH,
        scratch_types=[
            pltpu.VMEM((NCHQD, CHD), jnp.int32),
            pltpu.VMEM((CHD, DDEG), jnp.float32),
            pltpu.VMEM_SHARED((NPAD, DDEG), jnp.float32),
            [pltpu.SemaphoreType.DMA for _ in range(NBUF)],
        ],
    )
    def k(dst_hbm, out_hbm, dst_v, val_v, acc, ssem):
        c = lax.axis_index("c")
        s = lax.axis_index("s")
        wid = c * NSUB + s
        _fill_rows(val_v, CHD, DDEG, 0.0)

        @pl.loop(0, RPS // CHD)
        def _(i):
            pltpu.sync_copy(val_v, acc.at[pl.ds(s * RPS + i * CHD, CHD)])

        _fill_rows(val_v, CHD, DDEG, 1.0)
        plsc.subcore_barrier()

        @pl.loop(0, NCHD // NCHQD)
        def _(q):
            off = pl.multiple_of(wid * NCHD + q * NCHQD, NCHQD)
            pltpu.sync_copy(dst_hbm.at[pl.ds(off, NCHQD)], dst_v)
            for b in range(NBUF):
                pltpu.async_copy(
                    val_v, acc.at[dst_v.at[b]], ssem[b], add=True
                )

            @pl.loop(0, NCHQD - NBUF, step=NBUF)
            def _(k0):
                for b in range(NBUF):
                    pltpu.make_async_copy(
                        val_v, acc.at[dst_v.at[k0 + b]], ssem[b]
                    ).wait()
                    pltpu.async_copy(
                        val_v, acc.at[dst_v.at[k0 + NBUF + b]], ssem[b],
                        add=True,
                    )

            for b in range(NBUF):
                pltpu.make_async_copy(
                    val_v, acc.at[dst_v.at[NCHQD - NBUF + b]], ssem[b]
                ).wait()

        plsc.subcore_barrier()
        pltpu.sync_copy(
            acc.at[pl.ds(s * RPS, RPS)], out_hbm.at[c, pl.ds(s * RPS, RPS)]
        )

    return k(dst3)


def _sc_aggregate(table, src3, dst3, d):
    @functools.partial(
        pl.kernel,
        out_type=jax.ShapeDtypeStruct((NCORES, NPAD, d), jnp.float32),
        mesh=_MESH,
        scratch_types=[
            pltpu.VMEM((NCHQ, CH), jnp.int32),
            pltpu.VMEM((NCHQ, CH), jnp.int32),
            [pltpu.VMEM((CH, d), jnp.float32) for _ in range(NBUF)],
            pltpu.VMEM_SHARED((NPAD, d), jnp.float32),
            [pltpu.SemaphoreType.DMA for _ in range(NBUF)],
            [pltpu.SemaphoreType.DMA for _ in range(NBUF)],
        ],
    )
    def k(tab_hbm, src_hbm, dst_hbm, out_hbm, src_v, dst_v, bufs, acc,
          gsem, ssem):
        c = lax.axis_index("c")
        s = lax.axis_index("s")
        _fill_rows(bufs[0], CH, d, 0.0)

        @pl.loop(0, RPS // CH)
        def _(i):
            pltpu.sync_copy(bufs[0], acc.at[pl.ds(s * RPS + i * CH, CH)])

        plsc.subcore_barrier()

        n_my = jnp.where(c == 0, NCH0, NCH1)
        base = c * (NSUB * NCH0) + s * n_my

        @pl.loop(0, n_my // NCHQ)
        def _(q):
            off = pl.multiple_of(base + q * NCHQ, NCHQ)
            pltpu.sync_copy(src_hbm.at[pl.ds(off, NCHQ)], src_v)
            pltpu.sync_copy(dst_hbm.at[pl.ds(off, NCHQ)], dst_v)
            for b in range(NBUF):
                pltpu.async_copy(tab_hbm.at[src_v.at[b]], bufs[b], gsem[b])

            @pl.loop(0, NCHQ - NBUF, step=NBUF)
            def _(k0):
                for b in range(NBUF):
                    pltpu.make_async_copy(
                        tab_hbm.at[src_v.at[k0 + b]], bufs[b], gsem[b]
                    ).wait()
                    pltpu.async_copy(
                        bufs[b], acc.at[dst_v.at[k0 + b]], ssem[b], add=True
                    )
                for b in range(NBUF):
                    pltpu.make_async_copy(
                        bufs[b], acc.at[dst_v.at[k0 + b]], ssem[b]
                    ).wait()
                    pltpu.async_copy(
                        tab_hbm.at[src_v.at[k0 + NBUF + b]], bufs[b], gsem[b]
                    )

            for b in range(NBUF):
                pltpu.make_async_copy(
                    tab_hbm.at[src_v.at[NCHQ - NBUF + b]], bufs[b], gsem[b]
                ).wait()
                pltpu.async_copy(
                    bufs[b], acc.at[dst_v.at[NCHQ - NBUF + b]], ssem[b],
                    add=True,
                )
            for b in range(NBUF):
                pltpu.make_async_copy(
                    bufs[b], acc.at[dst_v.at[NCHQ - NBUF + b]], ssem[b]
                ).wait()

        plsc.subcore_barrier()
        pltpu.sync_copy(
            acc.at[pl.ds(s * RPS, RPS)], out_hbm.at[c, pl.ds(s * RPS, RPS)]
        )

    return k(table, src3, dst3)


_R = 1024


def _tc_matmul(x_p, W1):
    def body(x_ref, w_ref, h_ref):
        h_ref[...] = jnp.dot(
            x_ref[...], w_ref[...], preferred_element_type=jnp.float32
        )

    return pl.pallas_call(
        body,
        grid=(NPAD // _R,),
        in_specs=[
            pl.BlockSpec((_R, DIN), lambda i: (i, 0)),
            pl.BlockSpec((DIN, DH), lambda i: (0, 0)),
        ],
        out_specs=pl.BlockSpec((_R, DH), lambda i: (i, 0)),
        out_shape=jax.ShapeDtypeStruct((NPAD, DH), jnp.float32),
    )(x_p, W1)


def _tc_pre(h, p16):
    def body(h_ref, p_ref, hs_ref, dinv_ref):
        deg16 = p_ref[0, :, :DINV] + p_ref[1, :, :DINV] + 1.0
        dinv16 = lax.rsqrt(deg16)
        hs_ref[...] = h_ref[...] * dinv16[:, 0:1]
        dinv_ref[...] = dinv16

    return pl.pallas_call(
        body,
        grid=(NPAD // _R,),
        in_specs=[
            pl.BlockSpec((_R, DH), lambda i: (i, 0)),
            pl.BlockSpec((NCORES, _R, DDEG), lambda i: (0, i, 0)),
        ],
        out_specs=[
            pl.BlockSpec((_R, DH), lambda i: (i, 0)),
            pl.BlockSpec((_R, DINV), lambda i: (i, 0)),
        ],
        out_shape=[
            jax.ShapeDtypeStruct((NPAD, DH), jnp.float32),
            jax.ShapeDtypeStruct((NPAD, DINV), jnp.float32),
        ],
    )(h, p16)


def _tc_mid(p1, hs1, dinv16, W2p, b1r):
    def body(p_ref, hs_ref, dinv_ref, w_ref, b_ref, out_ref):
        dinv = dinv_ref[:, 0:1]
        agg = p_ref[0] + p_ref[1] + hs_ref[...]
        out1 = agg * dinv + b_ref[...]
        r = jnp.maximum(out1, 0.0)
        h2 = jnp.dot(r, w_ref[...], preferred_element_type=jnp.float32)
        out_ref[...] = h2 * dinv

    return pl.pallas_call(
        body,
        grid=(NPAD // _R,),
        in_specs=[
            pl.BlockSpec((NCORES, _R, DH), lambda i: (0, i, 0)),
            pl.BlockSpec((_R, DH), lambda i: (i, 0)),
            pl.BlockSpec((_R, DINV), lambda i: (i, 0)),
            pl.BlockSpec((DH, DOP), lambda i: (0, 0)),
            pl.BlockSpec((1, DH), lambda i: (0, 0)),
        ],
        out_specs=pl.BlockSpec((_R, DOP), lambda i: (i, 0)),
        out_shape=jax.ShapeDtypeStruct((NPAD, DOP), jnp.float32),
    )(p1, hs1, dinv16, W2p, b1r)


def _tc_post(p2, hs2, dinv16, b2r):
    def body(p_ref, hs_ref, dinv_ref, b_ref, out_ref):
        dinv = dinv_ref[:, 0:1]
        logits = (p_ref[0] + p_ref[1] + hs_ref[...]) * dinv + b_ref[...]
        colmask = lax.broadcasted_iota(jnp.int32, (_R, DOP), 1) < DOUT
        lm = jnp.where(colmask, logits, jnp.float32(-1e30))
        m = jnp.max(lm, axis=1, keepdims=True)
        e = jnp.where(colmask, jnp.exp(logits - m), 0.0)
        ssum = jnp.sum(e, axis=1, keepdims=True)
        out_ref[...] = logits - m - jnp.log(ssum)

    return pl.pallas_call(
        body,
        grid=(NPAD // _R,),
        in_specs=[
            pl.BlockSpec((NCORES, _R, DOP), lambda i: (0, i, 0)),
            pl.BlockSpec((_R, DOP), lambda i: (i, 0)),
            pl.BlockSpec((_R, DINV), lambda i: (i, 0)),
            pl.BlockSpec((1, DOP), lambda i: (0, 0)),
        ],
        out_specs=pl.BlockSpec((_R, DOP), lambda i: (i, 0)),
        out_shape=jax.ShapeDtypeStruct((NPAD, DOP), jnp.float32),
    )(p2, hs2, dinv16, b2r)


def kernel(x, edge_index, W1, b1, W2, b2):
    src = edge_index[0]
    dst = edge_index[1]
    srcp = jnp.concatenate([src, jnp.asarray(_PAD_SRC)]).reshape(TCH, CH)
    dstp = jnp.concatenate([dst, jnp.asarray(_PAD_DST)]).reshape(TCH, CH)
    x_p = jnp.pad(x, ((0, NPAD - N), (0, 0)))
    W2p = jnp.pad(W2, ((0, 0), (0, DOP - DOUT)))
    b2r = jnp.pad(b2, ((0, DOP - DOUT),)).reshape(1, DOP)
    b1r = b1.reshape(1, DH)

    h1 = _tc_matmul(x_p, W1)
    p16 = _sc_degree(dstp.reshape(TCHD, CHD))
    hs1, dinv16 = _tc_pre(h1, p16)
    p1 = _sc_aggregate(hs1, srcp, dstp, DH)
    hs2 = _tc_mid(p1, hs1, dinv16, W2p, b1r)
    p2 = _sc_aggregate(hs2, srcp, dstp, DOP)
    out = _tc_post(p2, hs2, dinv16, b2r)
    return out[:N, :DOUT]
